# Initial kernel scaffold; baseline (speedup 1.0000x reference)
import jax
import jax.numpy as jnp
from jax.experimental import pallas as pl


def kernel(Q_train, table, conv_w):
    raise NotImplementedError("write your pallas kernel here")


# speedup vs baseline: 19.1822x; 19.1822x over previous
import functools

import jax
import jax.numpy as jnp
from jax import lax
from jax.experimental import pallas as pl
from jax.experimental.pallas import tpu as pltpu
from jax.experimental.pallas import tpu_sc as plsc

B, F, D, V = 16384, 26, 32, 1000000
NC, NS = 2, 16
NW = NC * NS
BPW = B // NW
C = 64
G = BPW // C
RPC = C * F
KI = RPC // 128


def _body(q_hbm, table_hbm, w_hbm, out_hbm, idx_v, rows_v, w_v, out_v, sem):
    wid = lax.axis_index("s") * NC + lax.axis_index("c")
    pltpu.sync_copy(w_hbm, w_v)

    @pl.loop(0, G)
    def _chunk(g):
        row0 = wid * BPW + g * C
        pltpu.sync_copy(q_hbm.at[pl.ds(row0 * F, RPC)], idx_v)
        cps = [
            pltpu.async_copy(
                table_hbm.at[idx_v.at[pl.ds(j * 128, 128)]],
                rows_v.at[pl.ds(j * 128, 128)],
                sem,
            )
            for j in range(KI)
        ]
        for cp in cps:
            cp.wait()

        @pl.loop(0, C)
        def _row(b):
            base = b * F
            acc0 = rows_v[base, pl.ds(0, 16)] * w_v[0, pl.ds(0, 16)]
            acc1 = rows_v[base, pl.ds(16, 16)] * w_v[0, pl.ds(16, 16)]
            for f in range(1, F):
                acc0 = acc0 + rows_v[base + f, pl.ds(0, 16)] * w_v[f, pl.ds(0, 16)]
                acc1 = acc1 + rows_v[base + f, pl.ds(16, 16)] * w_v[f, pl.ds(16, 16)]
            out_v[b, pl.ds(0, 16)] = acc0
            out_v[b, pl.ds(16, 16)] = acc1

        pltpu.sync_copy(out_v, out_hbm.at[pl.ds(row0, C)])


@jax.jit
def _run(q2, table, w_exp):
    mesh = plsc.VectorSubcoreMesh(core_axis_name="c", subcore_axis_name="s")
    f = pl.kernel(
        _body,
        out_type=jax.ShapeDtypeStruct((B, D), jnp.float32),
        mesh=mesh,
        scratch_types=[
            pltpu.VMEM((RPC,), jnp.int32),
            pltpu.VMEM((RPC, D), jnp.float32),
            pltpu.VMEM((F, D), jnp.float32),
            pltpu.VMEM((C, D), jnp.float32),
            pltpu.SemaphoreType.DMA,
        ],
        compiler_params=pltpu.CompilerParams(use_tc_tiling_on_sc=False),
    )
    return f(q2, table, w_exp)


def kernel(Q_train, table, conv_w):
    q2 = Q_train.astype(jnp.int32).reshape(B * F)
    w_exp = jnp.broadcast_to(conv_w[0, 0, :, 0][:, None], (F, D))
    out = _run(q2, table, w_exp)
    return out[:, None, None, :]
